# baseline (speedup 1.0000x reference)
import functools

import jax
import jax.numpy as jnp
from jax.experimental import pallas as pl
from jax.experimental.pallas import tpu as pltpu

S, D, H, E, K = 2048, 768, 12, 8, 2
DH = D // H
FF = 4 * D


def _ln_qkv_body(x_ref, g_ref, b_ref, w_ref, bin_ref, qkv_ref):
    x = x_ref[...]
    m = jnp.mean(x, axis=-1, keepdims=True)
    v = jnp.mean((x - m) ** 2, axis=-1, keepdims=True)
    xn = (x - m) * jax.lax.rsqrt(v + 1e-5) * g_ref[...] + b_ref[...]
    qkv_ref[...] = jnp.dot(xn, w_ref[...], preferred_element_type=jnp.float32) + bin_ref[...]


def _ln_qkv(x, g, b, w_t, b_in, bs=256):
    return pl.pallas_call(
        _ln_qkv_body,
        grid=(S // bs,),
        in_specs=[
            pl.BlockSpec((bs, D), lambda i: (i, 0)),
            pl.BlockSpec((D,), lambda i: (0,)),
            pl.BlockSpec((D,), lambda i: (0,)),
            pl.BlockSpec((D, 3 * D), lambda i: (0, 0)),
            pl.BlockSpec((3 * D,), lambda i: (0,)),
        ],
        out_specs=pl.BlockSpec((bs, 3 * D), lambda i: (i, 0)),
        out_shape=jax.ShapeDtypeStruct((S, 3 * D), jnp.float32),
    )(x, g, b, w_t, b_in)


def _attn_body(q_ref, k_ref, v_ref, o_ref):
    q = q_ref[0]
    k = k_ref[0]
    v = v_ref[0]
    s = jnp.dot(q, k.T, preferred_element_type=jnp.float32) * (1.0 / (DH ** 0.5))
    m = jnp.max(s, axis=-1, keepdims=True)
    p = jnp.exp(s - m)
    p = p / jnp.sum(p, axis=-1, keepdims=True)
    o_ref[0] = jnp.dot(p, v, preferred_element_type=jnp.float32)


def _attention(q, k, v, bq=512):
    return pl.pallas_call(
        _attn_body,
        grid=(H, S // bq),
        in_specs=[
            pl.BlockSpec((1, bq, DH), lambda h, i: (h, i, 0)),
            pl.BlockSpec((1, S, DH), lambda h, i: (h, 0, 0)),
            pl.BlockSpec((1, S, DH), lambda h, i: (h, 0, 0)),
        ],
        out_specs=pl.BlockSpec((1, bq, DH), lambda h, i: (h, i, 0)),
        out_shape=jax.ShapeDtypeStruct((H, S, DH), jnp.float32),
    )(q, k, v)


def _proj_body(o_ref, src_ref, w_ref, b_ref, g_ref, bb_ref, wg_ref, x_ref, xn_ref, lg_ref):
    o = o_ref[...]
    x = jnp.dot(o, w_ref[...], preferred_element_type=jnp.float32) + b_ref[...] + src_ref[...]
    x_ref[...] = x
    m = jnp.mean(x, axis=-1, keepdims=True)
    v = jnp.mean((x - m) ** 2, axis=-1, keepdims=True)
    xn = (x - m) * jax.lax.rsqrt(v + 1e-5) * g_ref[...] + bb_ref[...]
    xn_ref[...] = xn
    lg_ref[...] = jnp.dot(xn, wg_ref[...], preferred_element_type=jnp.float32)


def _proj_ln2(o, src, w_out_t, b_out, g2, b2, wg_pad, bs=256):
    return pl.pallas_call(
        _proj_body,
        grid=(S // bs,),
        in_specs=[
            pl.BlockSpec((bs, D), lambda i: (i, 0)),
            pl.BlockSpec((bs, D), lambda i: (i, 0)),
            pl.BlockSpec((D, D), lambda i: (0, 0)),
            pl.BlockSpec((D,), lambda i: (0,)),
            pl.BlockSpec((D,), lambda i: (0,)),
            pl.BlockSpec((D,), lambda i: (0,)),
            pl.BlockSpec((D, 128), lambda i: (0, 0)),
        ],
        out_specs=[
            pl.BlockSpec((bs, D), lambda i: (i, 0)),
            pl.BlockSpec((bs, D), lambda i: (i, 0)),
            pl.BlockSpec((bs, 128), lambda i: (i, 0)),
        ],
        out_shape=[
            jax.ShapeDtypeStruct((S, D), jnp.float32),
            jax.ShapeDtypeStruct((S, D), jnp.float32),
            jax.ShapeDtypeStruct((S, 128), jnp.float32),
        ],
    )(o, src, w_out_t, b_out, g2, b2, wg_pad)


def _moe_body(xn_ref, w1_ref, b1_ref, w2_ref, b2_ref, gates_ref, out_ref):
    e = pl.program_id(1)
    xn = xn_ref[...]
    h = jnp.maximum(
        jnp.dot(xn, w1_ref[0], preferred_element_type=jnp.float32) + b1_ref[0, 0], 0.0)
    eo = jnp.dot(h, w2_ref[0], preferred_element_type=jnp.float32) + b2_ref[0, 0]
    gates = gates_ref[...]
    lane = jax.lax.broadcasted_iota(jnp.int32, gates.shape, 1)
    g = jnp.sum(jnp.where(lane == e, gates, 0.0), axis=1, keepdims=True)
    contrib = g * eo

    @pl.when(e == 0)
    def _():
        out_ref[...] = contrib

    @pl.when(e != 0)
    def _():
        out_ref[...] += contrib


def _moe_dense(xn, w1, b1, w2, b2, gates_pad, bt=256):
    return pl.pallas_call(
        _moe_body,
        grid=(S // bt, E),
        in_specs=[
            pl.BlockSpec((bt, D), lambda i, e: (i, 0)),
            pl.BlockSpec((1, D, FF), lambda i, e: (e, 0, 0)),
            pl.BlockSpec((1, 1, FF), lambda i, e: (e, 0, 0)),
            pl.BlockSpec((1, FF, D), lambda i, e: (e, 0, 0)),
            pl.BlockSpec((1, 1, D), lambda i, e: (e, 0, 0)),
            pl.BlockSpec((bt, 128), lambda i, e: (i, 0)),
        ],
        out_specs=pl.BlockSpec((bt, D), lambda i, e: (i, 0)),
        out_shape=jax.ShapeDtypeStruct((S, D), jnp.float32),
    )(xn, w1, b1, w2, b2, gates_pad)


def _add_body(a_ref, b_ref, o_ref):
    o_ref[...] = a_ref[...] + b_ref[...]


def _residual_add(a, b, bs=512):
    return pl.pallas_call(
        _add_body,
        grid=(S // bs,),
        in_specs=[
            pl.BlockSpec((bs, D), lambda i: (i, 0)),
            pl.BlockSpec((bs, D), lambda i: (i, 0)),
        ],
        out_specs=pl.BlockSpec((bs, D), lambda i: (i, 0)),
        out_shape=jax.ShapeDtypeStruct((S, D), jnp.float32),
    )(a, b)


def kernel(src, gamma1, beta1, W_in, b_in, W_out, b_out, gamma2, beta2, Wg, W1, b1, W2, b2):
    x0 = src.reshape(S, D)
    qkv = _ln_qkv(x0, gamma1, beta1, W_in.T, b_in)
    q = qkv[:, :D].reshape(S, H, DH).transpose(1, 0, 2)
    k = qkv[:, D:2 * D].reshape(S, H, DH).transpose(1, 0, 2)
    v = qkv[:, 2 * D:].reshape(S, H, DH).transpose(1, 0, 2)
    o = _attention(q, k, v).transpose(1, 0, 2).reshape(S, D)
    wg_pad = jnp.zeros((D, 128), jnp.float32).at[:, :E].set(Wg)
    x, xn, logits_pad = _proj_ln2(o, x0, W_out.T, b_out, gamma2, beta2, wg_pad)
    logits = logits_pad[:, :E]
    gates_all = jax.nn.softmax(logits, axis=-1)
    topw, topi = jax.lax.top_k(gates_all, K)
    topw = topw / topw.sum(-1, keepdims=True)
    gates = jnp.zeros((S, E), jnp.float32).at[jnp.arange(S)[:, None], topi].set(topw)
    gates_pad = jnp.zeros((S, 128), jnp.float32).at[:, :E].set(gates)
    moe = _moe_dense(xn, W1, b1.reshape(E, 1, FF), W2, b2.reshape(E, 1, D), gates_pad)
    out = _residual_add(x, moe)
    return (out.reshape(S, 1, D), gates)


# speedup vs baseline: 1.1307x; 1.1307x over previous
import functools

import jax
import jax.numpy as jnp
from jax import lax
from jax.experimental import pallas as pl
from jax.experimental.pallas import tpu as pltpu
from jax.experimental.pallas import tpu_sc as plsc

S, D, H, E, K = 2048, 768, 12, 8, 2
DH = D // H
FF = 4 * D
T = 256
NBMAX = 24
P = NBMAX * T
NW = 32


def _ln_qkv_body(x_ref, g_ref, b_ref, w_ref, bin_ref, qkv_ref):
    x = x_ref[...]
    m = jnp.mean(x, axis=-1, keepdims=True)
    v = jnp.mean((x - m) ** 2, axis=-1, keepdims=True)
    xn = (x - m) * lax.rsqrt(v + 1e-5) * g_ref[...] + b_ref[...]
    qkv_ref[...] = jnp.dot(xn, w_ref[...], preferred_element_type=jnp.float32) + bin_ref[...]


def _ln_qkv(x, g, b, w_t, b_in, bs=256):
    return pl.pallas_call(
        _ln_qkv_body,
        grid=(S // bs,),
        in_specs=[
            pl.BlockSpec((bs, D), lambda i: (i, 0)),
            pl.BlockSpec((D,), lambda i: (0,)),
            pl.BlockSpec((D,), lambda i: (0,)),
            pl.BlockSpec((D, 3 * D), lambda i: (0, 0)),
            pl.BlockSpec((3 * D,), lambda i: (0,)),
        ],
        out_specs=pl.BlockSpec((bs, 3 * D), lambda i: (i, 0)),
        out_shape=jax.ShapeDtypeStruct((S, 3 * D), jnp.float32),
    )(x, g, b, w_t, b_in)


def _attn_body(q_ref, k_ref, v_ref, o_ref):
    q = q_ref[0]
    k = k_ref[0]
    v = v_ref[0]
    s = jnp.dot(q, k.T, preferred_element_type=jnp.float32) * (1.0 / (DH ** 0.5))
    m = jnp.max(s, axis=-1, keepdims=True)
    p = jnp.exp(s - m)
    p = p / jnp.sum(p, axis=-1, keepdims=True)
    o_ref[0] = jnp.dot(p, v, preferred_element_type=jnp.float32)


def _attention(q, k, v, bq=512):
    return pl.pallas_call(
        _attn_body,
        grid=(H, S // bq),
        in_specs=[
            pl.BlockSpec((1, bq, DH), lambda h, i: (h, i, 0)),
            pl.BlockSpec((1, S, DH), lambda h, i: (h, 0, 0)),
            pl.BlockSpec((1, S, DH), lambda h, i: (h, 0, 0)),
        ],
        out_specs=pl.BlockSpec((1, bq, DH), lambda h, i: (h, i, 0)),
        out_shape=jax.ShapeDtypeStruct((H, S, DH), jnp.float32),
    )(q, k, v)


def _proj_body(o_ref, src_ref, w_ref, b_ref, g_ref, bb_ref, wg_ref, x_ref, xn_ref, lg_ref):
    o = o_ref[...]
    x = jnp.dot(o, w_ref[...], preferred_element_type=jnp.float32) + b_ref[...] + src_ref[...]
    x_ref[...] = x
    m = jnp.mean(x, axis=-1, keepdims=True)
    v = jnp.mean((x - m) ** 2, axis=-1, keepdims=True)
    xn = (x - m) * lax.rsqrt(v + 1e-5) * g_ref[...] + bb_ref[...]
    xn_ref[...] = xn
    lg_ref[...] = jnp.dot(xn, wg_ref[...], preferred_element_type=jnp.float32)


def _proj_ln2(o, src, w_out_t, b_out, g2, b2, wg_pad, bs=256):
    return pl.pallas_call(
        _proj_body,
        grid=(S // bs,),
        in_specs=[
            pl.BlockSpec((bs, D), lambda i: (i, 0)),
            pl.BlockSpec((bs, D), lambda i: (i, 0)),
            pl.BlockSpec((D, D), lambda i: (0, 0)),
            pl.BlockSpec((D,), lambda i: (0,)),
            pl.BlockSpec((D,), lambda i: (0,)),
            pl.BlockSpec((D,), lambda i: (0,)),
            pl.BlockSpec((D, 128), lambda i: (0, 0)),
        ],
        out_specs=[
            pl.BlockSpec((bs, D), lambda i: (i, 0)),
            pl.BlockSpec((bs, D), lambda i: (i, 0)),
            pl.BlockSpec((bs, 128), lambda i: (i, 0)),
        ],
        out_shape=[
            jax.ShapeDtypeStruct((S, D), jnp.float32),
            jax.ShapeDtypeStruct((S, D), jnp.float32),
            jax.ShapeDtypeStruct((S, 128), jnp.float32),
        ],
    )(o, src, w_out_t, b_out, g2, b2, wg_pad)


def _make_sc_gather(n_rows, table_rows, chunk):
    rows_per_worker = n_rows // NW
    nchunks = rows_per_worker // chunk
    mesh = plsc.VectorSubcoreMesh(core_axis_name="c", subcore_axis_name="s")

    @functools.partial(
        pl.kernel, mesh=mesh,
        out_type=jax.ShapeDtypeStruct((n_rows, D), jnp.float32),
        scratch_types=[
            pltpu.VMEM((chunk,), jnp.int32),
            pltpu.VMEM((chunk, D), jnp.float32),
            pltpu.SemaphoreType.DMA,
        ],
    )
    def gather(table_hbm, idx_hbm, out_hbm, idx_v, rows_v, sem):
        wid = lax.axis_index("s") * 2 + lax.axis_index("c")
        base = wid * rows_per_worker
        for c in range(nchunks):
            off = base + c * chunk
            pltpu.sync_copy(idx_hbm.at[pl.ds(off, chunk)], idx_v)
            pltpu.async_copy(table_hbm.at[idx_v], rows_v, sem).wait()
            pltpu.sync_copy(rows_v, out_hbm.at[pl.ds(off, chunk)])

    return gather


_sc_dispatch = _make_sc_gather(P, S, 96)
_sc_collect = _make_sc_gather(S * K, P, 64)


def _ffn_body(be_ref, xs_ref, w1_ref, b1_ref, w2_ref, b2_ref, ww_ref, out_ref):
    xs = xs_ref[...]
    h = jnp.maximum(
        jnp.dot(xs, w1_ref[0], preferred_element_type=jnp.float32) + b1_ref[0, 0], 0.0)
    y = jnp.dot(h, w2_ref[0], preferred_element_type=jnp.float32) + b2_ref[0, 0]
    out_ref[...] = y * ww_ref[...]


def _moe_ffn(block_expert, xs, w1, b1, w2, b2, row_w):
    grid_spec = pltpu.PrefetchScalarGridSpec(
        num_scalar_prefetch=1,
        grid=(NBMAX,),
        in_specs=[
            pl.BlockSpec((T, D), lambda b, be: (b, 0)),
            pl.BlockSpec((1, D, FF), lambda b, be: (be[b], 0, 0)),
            pl.BlockSpec((1, 1, FF), lambda b, be: (be[b], 0, 0)),
            pl.BlockSpec((1, FF, D), lambda b, be: (be[b], 0, 0)),
            pl.BlockSpec((1, 1, D), lambda b, be: (be[b], 0, 0)),
            pl.BlockSpec((T, 1), lambda b, be: (b, 0)),
        ],
        out_specs=pl.BlockSpec((T, D), lambda b, be: (b, 0)),
    )
    return pl.pallas_call(
        _ffn_body,
        grid_spec=grid_spec,
        out_shape=jax.ShapeDtypeStruct((P, D), jnp.float32),
    )(block_expert, xs, w1, b1, w2, b2, row_w)


def _combine_body(x_ref, yg_ref, o_ref):
    o_ref[...] = x_ref[...] + yg_ref[:, 0, :] + yg_ref[:, 1, :]


def _combine(x, yg, bs=512):
    return pl.pallas_call(
        _combine_body,
        grid=(S // bs,),
        in_specs=[
            pl.BlockSpec((bs, D), lambda i: (i, 0)),
            pl.BlockSpec((bs, K, D), lambda i: (i, 0, 0)),
        ],
        out_specs=pl.BlockSpec((bs, D), lambda i: (i, 0)),
        out_shape=jax.ShapeDtypeStruct((S, D), jnp.float32),
    )(x, yg)


def kernel(src, gamma1, beta1, W_in, b_in, W_out, b_out, gamma2, beta2, Wg, W1, b1, W2, b2):
    x0 = src.reshape(S, D)
    qkv = _ln_qkv(x0, gamma1, beta1, W_in.T, b_in)
    q = qkv[:, :D].reshape(S, H, DH).transpose(1, 0, 2)
    k = qkv[:, D:2 * D].reshape(S, H, DH).transpose(1, 0, 2)
    v = qkv[:, 2 * D:].reshape(S, H, DH).transpose(1, 0, 2)
    o = _attention(q, k, v).transpose(1, 0, 2).reshape(S, D)
    wg_pad = jnp.zeros((D, 128), jnp.float32).at[:, :E].set(Wg)
    x, xn, logits_pad = _proj_ln2(o, x0, W_out.T, b_out, gamma2, beta2, wg_pad)

    logits = logits_pad[:, :E]
    gates_all = jax.nn.softmax(logits, axis=-1)
    topw, topi = lax.top_k(gates_all, K)
    topw = topw / topw.sum(-1, keepdims=True)
    gates = jnp.zeros((S, E), jnp.float32).at[jnp.arange(S)[:, None], topi].set(topw)

    e_pair = topi.reshape(-1)
    w_pair = topw.reshape(-1)
    onehot = (e_pair[:, None] == jnp.arange(E)[None, :]).astype(jnp.int32)
    rank = jnp.cumsum(onehot, axis=0) - 1
    rank_j = jnp.sum(rank * onehot, axis=1)
    counts = jnp.sum(onehot, axis=0)
    nblk = (counts + T - 1) // T
    blk_start = jnp.cumsum(nblk) - nblk
    dst = blk_start[e_pair] * T + rank_j
    tok_pair = jnp.arange(S * K, dtype=jnp.int32) // K
    row_token = jnp.zeros((P,), jnp.int32).at[dst].set(tok_pair)
    row_w = jnp.zeros((P, 1), jnp.float32).at[dst, 0].set(w_pair)
    block_expert = jnp.sum(
        (jnp.arange(NBMAX, dtype=jnp.int32)[:, None] >= blk_start[None, :]).astype(jnp.int32),
        axis=1) - 1

    xs = _sc_dispatch(xn, row_token)
    ys = _moe_ffn(block_expert, xs, W1, b1.reshape(E, 1, FF), W2, b2.reshape(E, 1, D), row_w)
    yg = _sc_collect(ys, dst.astype(jnp.int32))
    out = _combine(x, yg.reshape(S, K, D))
    return (out.reshape(S, 1, D), gates)
